# baseline (speedup 1.0000x reference)
import functools

import jax
import jax.numpy as jnp
from jax import lax
from jax.experimental import pallas as pl
from jax.experimental.pallas import tpu as pltpu
from jax.experimental.pallas import tpu_sc as plsc

_NUM_POS = 8192
_EMB = 1024


@functools.cache
def _copy_kernel():
    info = plsc.get_sparse_core_info()
    nc, ns = info.num_cores, info.num_subcores
    nw = nc * ns
    rows_per_w = _NUM_POS // nw
    mesh = plsc.VectorSubcoreMesh(core_axis_name="c", subcore_axis_name="s")

    @functools.partial(
        pl.kernel,
        mesh=mesh,
        out_type=jax.ShapeDtypeStruct((_NUM_POS, _EMB), jnp.float32),
    )
    def copy_k(table_hbm, out_hbm):
        wid = lax.axis_index("s") * nc + lax.axis_index("c")
        base = wid * rows_per_w
        pltpu.sync_copy(table_hbm.at[pl.ds(base, rows_per_w)],
                        out_hbm.at[pl.ds(base, rows_per_w)])

    return copy_k


def kernel(x, pos_embed_weight):
    del x
    out = _copy_kernel()(pos_embed_weight)
    return out[None]


# speedup vs baseline: 23.3377x; 23.3377x over previous
import functools

import jax
import jax.numpy as jnp
from jax import lax
from jax.experimental import pallas as pl
from jax.experimental.pallas import tpu as pltpu
from jax.experimental.pallas import tpu_sc as plsc

_NUM_POS = 8192
_EMB = 1024
_CHUNK = 32
_NBUF = 2


@functools.cache
def _copy_kernel():
    info = plsc.get_sparse_core_info()
    nc, ns = info.num_cores, info.num_subcores
    nw = nc * ns
    rows_per_w = _NUM_POS // nw
    nchunks = rows_per_w // _CHUNK
    mesh = plsc.VectorSubcoreMesh(core_axis_name="c", subcore_axis_name="s")

    @functools.partial(
        pl.kernel,
        mesh=mesh,
        out_type=jax.ShapeDtypeStruct((_NUM_POS, _EMB), jnp.float32),
        scratch_types=[
            pltpu.VMEM((_NBUF, _CHUNK, _EMB), jnp.float32),
            pltpu.SemaphoreType.DMA,
            pltpu.SemaphoreType.DMA,
            pltpu.SemaphoreType.DMA,
            pltpu.SemaphoreType.DMA,
        ],
    )
    def copy_k(table_hbm, out_hbm, buf, si0, si1, so0, so1):
        sin = (si0, si1)
        sout = (so0, so1)
        wid = lax.axis_index("s") * nc + lax.axis_index("c")
        base = wid * rows_per_w
        hin = [None] * nchunks
        hout = [None] * nchunks
        for i in range(nchunks):
            b = i % _NBUF
            off = base + i * _CHUNK
            if i >= _NBUF:
                hout[i - _NBUF].wait()
            hin[i] = pltpu.async_copy(
                table_hbm.at[pl.ds(off, _CHUNK)], buf.at[b], sin[b])
            hin[i].wait()
            hout[i] = pltpu.async_copy(
                buf.at[b], out_hbm.at[pl.ds(off, _CHUNK)], sout[b])
        for i in range(nchunks - _NBUF, nchunks):
            hout[i].wait()

    return copy_k


def kernel(x, pos_embed_weight):
    del x
    out = _copy_kernel()(pos_embed_weight)
    return out[None]


# speedup vs baseline: 24.2194x; 1.0378x over previous
import functools

import jax
import jax.numpy as jnp
from jax import lax
from jax.experimental import pallas as pl
from jax.experimental.pallas import tpu as pltpu
from jax.experimental.pallas import tpu_sc as plsc

_NUM_POS = 8192
_EMB = 1024
_CHUNK = 32
_NBUF = 3


@functools.cache
def _copy_kernel():
    info = plsc.get_sparse_core_info()
    nc, ns = info.num_cores, info.num_subcores
    nw = nc * ns
    rows_per_w = _NUM_POS // nw
    nchunks = rows_per_w // _CHUNK
    mesh = plsc.VectorSubcoreMesh(core_axis_name="c", subcore_axis_name="s")

    @functools.partial(
        pl.kernel,
        mesh=mesh,
        out_type=jax.ShapeDtypeStruct((_NUM_POS, _EMB), jnp.float32),
        scratch_types=[
            pltpu.VMEM((_NBUF, _CHUNK, _EMB), jnp.float32),
            pltpu.SemaphoreType.DMA,
            pltpu.SemaphoreType.DMA,
            pltpu.SemaphoreType.DMA,
            pltpu.SemaphoreType.DMA,
            pltpu.SemaphoreType.DMA,
            pltpu.SemaphoreType.DMA,
        ],
    )
    def copy_k(table_hbm, out_hbm, buf, si0, si1, si2, so0, so1, so2):
        sin = (si0, si1, si2)
        sout = (so0, so1, so2)
        wid = lax.axis_index("s") * nc + lax.axis_index("c")
        base = wid * rows_per_w
        hin = [None] * nchunks
        hout = [None] * nchunks

        def start_in(i):
            b = i % _NBUF
            if i >= _NBUF:
                hout[i - _NBUF].wait()
            hin[i] = pltpu.async_copy(
                table_hbm.at[pl.ds(base + i * _CHUNK, _CHUNK)],
                buf.at[b], sin[b])

        start_in(0)
        for i in range(nchunks):
            if i + 1 < nchunks:
                start_in(i + 1)
            b = i % _NBUF
            hin[i].wait()
            hout[i] = pltpu.async_copy(
                buf.at[b], out_hbm.at[pl.ds(base + i * _CHUNK, _CHUNK)],
                sout[b])
        for i in range(nchunks - _NBUF, nchunks):
            hout[i].wait()

    return copy_k


def kernel(x, pos_embed_weight):
    del x
    out = _copy_kernel()(pos_embed_weight)
    return out[None]
